# baseline (speedup 1.0000x reference)
import functools

import jax
import jax.numpy as jnp
from jax import lax
from jax.experimental import pallas as pl
from jax.experimental.pallas import tpu as pltpu
from jax.experimental.pallas import tpu_sc as plsc

_VT = 4096
_BT = 16
_NEG = -1e30


def _gather_embeddings(emb8, idx_flat, d):
    info = plsc.get_sparse_core_info()
    nw = info.num_cores * info.num_subcores
    n = idx_flat.shape[0]
    per_w = n // nw
    rpg = 128 // d
    sh = rpg.bit_length() - 1
    chunks = per_w // 16
    mesh = plsc.VectorSubcoreMesh(core_axis_name="c", subcore_axis_name="s")

    @functools.partial(
        pl.kernel,
        mesh=mesh,
        out_type=jax.ShapeDtypeStruct((n, 128), emb8.dtype),
        scratch_types=[
            pltpu.VMEM((per_w,), jnp.int32),
            pltpu.VMEM((per_w,), jnp.int32),
            pltpu.VMEM((per_w, 128), emb8.dtype),
            pltpu.SemaphoreType.DMA,
        ],
    )
    def gather(table_hbm, idx_hbm, out_hbm, idx_v, g_v, rows_v, sem):
        wid = lax.axis_index("s") * info.num_cores + lax.axis_index("c")
        base = wid * per_w
        pltpu.sync_copy(idx_hbm.at[pl.ds(base, per_w)], idx_v)
        for c in range(chunks):
            g_v[pl.ds(c * 16, 16)] = jnp.right_shift(idx_v[pl.ds(c * 16, 16)], sh)
        pltpu.async_copy(table_hbm.at[g_v], rows_v, sem).wait()
        pltpu.sync_copy(rows_v, out_hbm.at[pl.ds(base, per_w)])

    return gather(emb8, idx_flat)


def _select_body(idx_ref, e8_ref, out_ref):
    d = out_ref.shape[1]
    rpg = 128 // d
    sub = jnp.bitwise_and(idx_ref[...], rpg - 1)
    acc = jnp.zeros(out_ref.shape, jnp.float32)
    for s in range(rpg):
        m = (sub == s).astype(jnp.float32)
        acc = acc + m * e8_ref[:, s * d:(s + 1) * d]
    out_ref[...] = acc


def _lse_body(e_ref, wt_ref, lse_ref, m_scr, s_scr, l1_scr, *, kdim):
    j = pl.program_id(0)
    nj = pl.num_programs(0)

    @pl.when(j == 0)
    def _init():
        l1_scr[...] = jnp.sum(
            jnp.abs(e_ref[:, :kdim].astype(jnp.float32)), axis=1, keepdims=True)
        m_scr[...] = jnp.full(m_scr.shape, _NEG, jnp.float32)
        s_scr[...] = jnp.zeros(s_scr.shape, jnp.float32)

    logits = lax.dot_general(e_ref[...], wt_ref[...],
                             (((1,), (0,)), ((), ())),
                             preferred_element_type=jnp.float32)
    wmax = jnp.max(jnp.abs(wt_ref[:kdim, :].astype(jnp.float32)))
    bmax = jnp.max(wt_ref[kdim:kdim + 1, :].astype(jnp.float32))
    m_j = l1_scr[...] * wmax + bmax
    m_prev = m_scr[...]
    m_new = jnp.maximum(m_prev, m_j)
    s_scr[...] = (s_scr[...] * jnp.exp(m_prev - m_new)
                  + jnp.sum(jnp.exp(logits - m_new), axis=1, keepdims=True))
    m_scr[...] = m_new

    @pl.when(j == nj - 1)
    def _fin():
        lse_ref[...] = m_scr[...] + jnp.log(s_scr[...])


def _out_body(e_ref, wt_ref, lse_ref, y_hbm, buf, sem, *, nbuf):
    i = pl.program_id(0)
    ni = pl.num_programs(0)
    bt, v = buf.shape[1], buf.shape[2]
    logits = lax.dot_general(e_ref[...], wt_ref[...],
                             (((1,), (0,)), ((), ())),
                             preferred_element_type=jnp.float32)
    val = logits[:, :v] - lse_ref[...]
    slot = lax.rem(i, nbuf)

    @pl.when(i >= nbuf)
    def _drain_prev():
        pltpu.make_async_copy(buf.at[slot], y_hbm.at[pl.ds(0, bt)],
                              sem.at[slot]).wait()

    buf[slot] = val
    pltpu.make_async_copy(buf.at[slot], y_hbm.at[pl.ds(i * bt, bt)],
                          sem.at[slot]).start()

    @pl.when(i == ni - 1)
    def _drain_all():
        for s in range(nbuf):
            pltpu.make_async_copy(buf.at[s], y_hbm.at[pl.ds(0, bt)],
                                  sem.at[s]).wait()


def kernel(x, emb, W, b):
    bsz, ctx = x.shape
    v, d = emb.shape
    k = ctx * d

    idx = x.reshape(-1).astype(jnp.int32)
    n = bsz * ctx
    emb8 = emb.reshape(v * d // 128, 128)
    e8 = _gather_embeddings(emb8, idx, d)
    esel = pl.pallas_call(
        _select_body,
        in_specs=[
            pl.BlockSpec((n, 1), lambda: (0, 0)),
            pl.BlockSpec((n, 128), lambda: (0, 0)),
        ],
        out_specs=pl.BlockSpec((n, d), lambda: (0, 0)),
        out_shape=jax.ShapeDtypeStruct((n, d), jnp.float32),
    )(idx.reshape(n, 1), e8)
    e = esel.reshape(bsz, k)

    nj = pl.cdiv(v, _VT)
    vpad = nj * _VT
    kp = ((k + 1 + 15) // 16) * 16
    wt = jnp.concatenate([
        jnp.pad(W, ((0, vpad - v), (0, 0))).T,
        jnp.pad(b, (0, vpad - v), constant_values=_NEG).reshape(1, vpad),
        jnp.zeros((kp - k - 1, vpad), jnp.float32),
    ], axis=0).astype(jnp.bfloat16)
    ep = jnp.concatenate([
        e.astype(jnp.bfloat16),
        jnp.ones((bsz, 1), jnp.bfloat16),
        jnp.zeros((bsz, kp - k - 1), jnp.bfloat16),
    ], axis=1)

    lse = pl.pallas_call(
        functools.partial(_lse_body, kdim=k),
        grid=(nj,),
        in_specs=[
            pl.BlockSpec((bsz, kp), lambda j: (0, 0)),
            pl.BlockSpec((kp, _VT), lambda j: (0, j)),
        ],
        out_specs=pl.BlockSpec((bsz, 1), lambda j: (0, 0)),
        out_shape=jax.ShapeDtypeStruct((bsz, 1), jnp.float32),
        scratch_shapes=[
            pltpu.VMEM((bsz, 1), jnp.float32),
            pltpu.VMEM((bsz, 1), jnp.float32),
            pltpu.VMEM((bsz, 1), jnp.float32),
        ],
    )(ep, wt)

    nbuf = 4
    y = pl.pallas_call(
        functools.partial(_out_body, nbuf=nbuf),
        grid=(bsz // _BT,),
        in_specs=[
            pl.BlockSpec((_BT, kp), lambda i: (i, 0)),
            pl.BlockSpec((kp, vpad), lambda i: (0, 0)),
            pl.BlockSpec((_BT, 1), lambda i: (i, 0)),
        ],
        out_specs=pl.BlockSpec(memory_space=pltpu.MemorySpace.HBM),
        out_shape=jax.ShapeDtypeStruct((bsz, v), jnp.float32),
        scratch_shapes=[
            pltpu.VMEM((nbuf, _BT, v), jnp.float32),
            pltpu.SemaphoreType.DMA((nbuf,)),
        ],
    )(ep, wt, lse)
    return y


# speedup vs baseline: 1.0135x; 1.0135x over previous
import functools

import jax
import jax.numpy as jnp
from jax import lax
from jax.experimental import pallas as pl
from jax.experimental.pallas import tpu as pltpu
from jax.experimental.pallas import tpu_sc as plsc

_VT = 4096
_BT = 32
_NEG = -1e30


def _gather_embeddings(emb8, idx_flat, d):
    info = plsc.get_sparse_core_info()
    nw = info.num_cores * info.num_subcores
    n = idx_flat.shape[0]
    per_w = n // nw
    rpg = 128 // d
    sh = rpg.bit_length() - 1
    chunks = per_w // 16
    mesh = plsc.VectorSubcoreMesh(core_axis_name="c", subcore_axis_name="s")

    @functools.partial(
        pl.kernel,
        mesh=mesh,
        out_type=jax.ShapeDtypeStruct((n, 128), emb8.dtype),
        scratch_types=[
            pltpu.VMEM((per_w,), jnp.int32),
            pltpu.VMEM((per_w,), jnp.int32),
            pltpu.VMEM((per_w, 128), emb8.dtype),
            pltpu.SemaphoreType.DMA,
        ],
    )
    def gather(table_hbm, idx_hbm, out_hbm, idx_v, g_v, rows_v, sem):
        wid = lax.axis_index("s") * info.num_cores + lax.axis_index("c")
        base = wid * per_w
        pltpu.sync_copy(idx_hbm.at[pl.ds(base, per_w)], idx_v)
        for c in range(chunks):
            g_v[pl.ds(c * 16, 16)] = jnp.right_shift(idx_v[pl.ds(c * 16, 16)], sh)
        pltpu.async_copy(table_hbm.at[g_v], rows_v, sem).wait()
        pltpu.sync_copy(rows_v, out_hbm.at[pl.ds(base, per_w)])

    return gather(emb8, idx_flat)


def _select_body(idx_ref, e8_ref, out_ref):
    d = out_ref.shape[1]
    rpg = 128 // d
    sub = jnp.bitwise_and(idx_ref[...], rpg - 1)
    acc = jnp.zeros(out_ref.shape, jnp.float32)
    for s in range(rpg):
        m = (sub == s).astype(jnp.float32)
        acc = acc + m * e8_ref[:, s * d:(s + 1) * d]
    out_ref[...] = acc


def _lse_body(e_ref, wt_ref, lse_ref, m_scr, s_scr, l1_scr, *, kdim):
    j = pl.program_id(0)
    nj = pl.num_programs(0)

    @pl.when(j == 0)
    def _init():
        l1_scr[...] = jnp.sum(
            jnp.abs(e_ref[:, :kdim].astype(jnp.float32)), axis=1, keepdims=True)
        m_scr[...] = jnp.full(m_scr.shape, _NEG, jnp.float32)
        s_scr[...] = jnp.zeros(s_scr.shape, jnp.float32)

    logits = lax.dot_general(e_ref[...], wt_ref[...],
                             (((1,), (0,)), ((), ())),
                             preferred_element_type=jnp.float32)
    wmax = jnp.max(jnp.abs(wt_ref[:kdim, :].astype(jnp.float32)))
    bmax = jnp.max(wt_ref[kdim:kdim + 1, :].astype(jnp.float32))
    m_j = l1_scr[...] * wmax + bmax
    m_prev = m_scr[...]
    m_new = jnp.maximum(m_prev, m_j)
    s_scr[...] = (s_scr[...] * jnp.exp(m_prev - m_new)
                  + jnp.sum(jnp.exp(logits - m_new), axis=1, keepdims=True))
    m_scr[...] = m_new

    @pl.when(j == nj - 1)
    def _fin():
        lse_ref[...] = m_scr[...] + jnp.log(s_scr[...])


def _out_body(e_ref, wt_ref, lse_ref, y_ref):
    logits = lax.dot_general(e_ref[...], wt_ref[...],
                             (((1,), (0,)), ((), ())),
                             preferred_element_type=jnp.float32)
    y_ref[...] = logits - lse_ref[...]


def kernel(x, emb, W, b):
    bsz, ctx = x.shape
    v, d = emb.shape
    k = ctx * d

    idx = x.reshape(-1).astype(jnp.int32)
    n = bsz * ctx
    emb8 = emb.reshape(v * d // 128, 128)
    e8 = _gather_embeddings(emb8, idx, d)
    esel = pl.pallas_call(
        _select_body,
        in_specs=[
            pl.BlockSpec((n, 1), lambda: (0, 0)),
            pl.BlockSpec((n, 128), lambda: (0, 0)),
        ],
        out_specs=pl.BlockSpec((n, d), lambda: (0, 0)),
        out_shape=jax.ShapeDtypeStruct((n, d), jnp.float32),
    )(idx.reshape(n, 1), e8)
    e = esel.reshape(bsz, k)

    nj = pl.cdiv(v, _VT)
    vpad = nj * _VT
    kp = ((k + 1 + 15) // 16) * 16
    wt = jnp.concatenate([
        jnp.pad(W, ((0, vpad - v), (0, 0))).T,
        jnp.pad(b, (0, vpad - v), constant_values=_NEG).reshape(1, vpad),
        jnp.zeros((kp - k - 1, vpad), jnp.float32),
    ], axis=0).astype(jnp.bfloat16)
    ep = jnp.concatenate([
        e.astype(jnp.bfloat16),
        jnp.ones((bsz, 1), jnp.bfloat16),
        jnp.zeros((bsz, kp - k - 1), jnp.bfloat16),
    ], axis=1)

    lse = pl.pallas_call(
        functools.partial(_lse_body, kdim=k),
        grid=(nj,),
        in_specs=[
            pl.BlockSpec((bsz, kp), lambda j: (0, 0)),
            pl.BlockSpec((kp, _VT), lambda j: (0, j)),
        ],
        out_specs=pl.BlockSpec((bsz, 1), lambda j: (0, 0)),
        out_shape=jax.ShapeDtypeStruct((bsz, 1), jnp.float32),
        scratch_shapes=[
            pltpu.VMEM((bsz, 1), jnp.float32),
            pltpu.VMEM((bsz, 1), jnp.float32),
            pltpu.VMEM((bsz, 1), jnp.float32),
        ],
    )(ep, wt)

    y = pl.pallas_call(
        _out_body,
        grid=(bsz // _BT,),
        in_specs=[
            pl.BlockSpec((_BT, kp), lambda i: (i, 0)),
            pl.BlockSpec((kp, vpad), lambda i: (0, 0)),
            pl.BlockSpec((_BT, 1), lambda i: (i, 0)),
        ],
        out_specs=pl.BlockSpec((_BT, vpad), lambda i: (i, 0)),
        out_shape=jax.ShapeDtypeStruct((bsz, v), jnp.float32),
    )(ep, wt, lse)
    return y
